# baseline (speedup 1.0000x reference)
import functools

import jax
import jax.numpy as jnp
from jax import lax
from jax.experimental import pallas as pl
from jax.experimental.pallas import tpu as pltpu
from jax.experimental.pallas import tpu_sc as plsc

_N = 4096
_M = 4096
_E = 64
_B = 8192

_NC = 2
_NS = 16
_NW = _NC * _NS
_RPW = _B // _NW
_G = 2
_NGP = _RPW // _G
_ECH = _E // 16
_GCH = _G * _ECH
_GW = 128
_NGD = _RPW * _E // _GW
_GPW = _G * _E


def _body(x_hbm, ind_hbm, val_hbm, bias_hbm, out_hbm,
          ind_v, val_v, bias_v, biasd_v, idx_v, xg_v, buf0, buf1,
          sem_g, sem0, sem1, sem_b, sem_h):
    cid = lax.axis_index("c")
    sid = lax.axis_index("s")
    wid = sid * _NC + cid
    base = wid * _RPW

    pltpu.make_async_copy(val_hbm, val_v, sem_b).start()
    pltpu.make_async_copy(bias_hbm, bias_v, sem_b).start()
    for r in range(_G):
        pltpu.make_async_copy(
            bias_hbm, buf0.at[pl.ds(r * _M, _M)], sem_b).start()
        pltpu.make_async_copy(
            bias_hbm, buf1.at[pl.ds(r * _M, _M)], sem1).start()
    pltpu.sync_copy(ind_hbm, ind_v)

    def mk_idx(r, _):
        rb = (base + r) * _N
        for kk in range(_ECH):
            src_c = ind_v[0, pl.ds(kk * 16, 16)]
            idx_v[pl.ds(r * _E + kk * 16, 16)] = src_c + jnp.full(
                (16,), rb, jnp.int32)
        return 0

    lax.fori_loop(0, _RPW, mk_idx, 0)

    def g_copy(off, sem):
        return pltpu.make_async_copy(
            x_hbm.at[idx_v.at[pl.ds(off, _GW)]],
            xg_v.at[pl.ds(off, _GW)], sem)

    def g_fire(g, sem):
        for c in range(_GPW // _GW):
            g_copy(g * _GPW + c * _GW, sem).start()

    def g_drain(sem):
        for c in range(_GPW // _GW):
            g_copy(0, sem).wait()

    g_fire(0, sem_g)
    g_fire(1, sem_h)

    pltpu.make_async_copy(val_hbm, val_v, sem_b).wait()
    pltpu.make_async_copy(bias_hbm, bias_v, sem_b).wait()
    for r in range(_G):
        pltpu.make_async_copy(
            bias_hbm, buf0.at[pl.ds(0, _M)], sem_b).wait()

    for kk in range(_ECH):
        dst_c = ind_v[1, pl.ds(kk * 16, 16)]
        biasd_v[pl.ds(kk * 16, 16)] = plsc.load_gather(bias_v, [dst_c])

    def scatter_group(buf, g):
        for k in range(_GCH):
            rl = k // _ECH
            kk = k % _ECH
            fc = ind_v[1, pl.ds(kk * 16, 16)] + jnp.full(
                (16,), rl * _M, jnp.int32)
            bd = biasd_v[pl.ds(kk * 16, 16)]
            cc = (xg_v[pl.ds(g * (_G * _E) + k * 16, 16)]
                  * val_v[pl.ds(kk * 16, 16)])
            plsc.store_scatter(buf, [fc], bd)
            plsc.addupdate_scatter(buf, [fc], cc)

    def group_pair(gg, _):
        g0 = 2 * gg
        g1 = 2 * gg + 1

        @pl.when(gg > 0)
        def _wait0():
            pltpu.make_async_copy(
                buf0, out_hbm.at[pl.ds(base * _M, _G * _M)], sem0).wait()

        g_drain(sem_g)
        scatter_group(buf0, g0)
        pltpu.make_async_copy(
            buf0, out_hbm.at[pl.ds((base + g0 * _G) * _M, _G * _M)],
            sem0).start()

        @pl.when(gg < _NGP // 2 - 1)
        def _pf0():
            g_fire(g0 + 2, sem_g)

        @pl.when(gg > 0)
        def _wait1():
            pltpu.make_async_copy(
                buf1, out_hbm.at[pl.ds(base * _M, _G * _M)], sem1).wait()

        @pl.when(gg == 0)
        def _wait1_fills():
            for r in range(_G):
                pltpu.make_async_copy(
                    bias_hbm, buf1.at[pl.ds(0, _M)], sem1).wait()

        g_drain(sem_h)
        scatter_group(buf1, g1)
        pltpu.make_async_copy(
            buf1, out_hbm.at[pl.ds((base + g1 * _G) * _M, _G * _M)],
            sem1).start()

        @pl.when(gg < _NGP // 2 - 1)
        def _pf1():
            g_fire(g1 + 2, sem_h)

        return 0

    lax.fori_loop(0, _NGP // 2, group_pair, 0)

    pltpu.make_async_copy(
        buf0, out_hbm.at[pl.ds(base * _M, _G * _M)], sem0).wait()
    pltpu.make_async_copy(
        buf1, out_hbm.at[pl.ds(base * _M, _G * _M)], sem1).wait()


@functools.partial(jax.jit, static_argnames=())
def _sc_sparse_linear(x_flat, indices, values, bias_flat):
    mesh = plsc.VectorSubcoreMesh(core_axis_name="c", subcore_axis_name="s")
    kfn = functools.partial(
        pl.kernel, mesh=mesh,
        compiler_params=pltpu.CompilerParams(needs_layout_passes=False),
        out_type=jax.ShapeDtypeStruct((_B * _M,), jnp.float32),
        scratch_types=[
            pltpu.VMEM((2, _E), jnp.int32),
            pltpu.VMEM((_E,), jnp.float32),
            pltpu.VMEM((_M,), jnp.float32),
            pltpu.VMEM((_E,), jnp.float32),
            pltpu.VMEM((_RPW * _E,), jnp.int32),
            pltpu.VMEM((_RPW * _E,), jnp.float32),
            pltpu.VMEM((_G * _M,), jnp.float32),
            pltpu.VMEM((_G * _M,), jnp.float32),
            pltpu.SemaphoreType.DMA,
            pltpu.SemaphoreType.DMA,
            pltpu.SemaphoreType.DMA,
            pltpu.SemaphoreType.DMA,
            pltpu.SemaphoreType.DMA,
        ],
    )(_body)
    return kfn(x_flat, indices, values, bias_flat)


def kernel(x, indices, values, bias):
    Bn, Nn, _ = x.shape
    Mn = bias.shape[0]
    x_flat = x.reshape(Bn * Nn)
    out = _sc_sparse_linear(
        x_flat, indices.astype(jnp.int32), values, bias.reshape(Mn))
    return out.reshape(Bn, Mn, 1)


# speedup vs baseline: 1.0255x; 1.0255x over previous
import functools

import jax
import jax.numpy as jnp
from jax import lax
from jax.experimental import pallas as pl
from jax.experimental.pallas import tpu as pltpu
from jax.experimental.pallas import tpu_sc as plsc

_N = 4096
_M = 4096
_E = 64
_B = 8192

_NC = 2
_NS = 16
_NW = _NC * _NS
_RPW = _B // _NW
_G = 4
_NBUF = 4
_NGP = _RPW // _G
_NIT = _NGP // _NBUF
_ECH = _E // 16
_GCH = _G * _ECH
_GW = 128
_GPW = _G * _E


def _body(x_hbm, ind_hbm, val_hbm, bias_hbm, out_hbm,
          ind_v, val_v, bias_v, biasd_v, idx_v, xg_v,
          b0, b1, b2, b3, os0, os1, os2, os3, gs0, gs1, gs2, gs3, sem_b):
    bufs = [b0, b1, b2, b3]
    osems = [os0, os1, os2, os3]
    gsems = [gs0, gs1, gs2, gs3]
    cid = lax.axis_index("c")
    sid = lax.axis_index("s")
    wid = sid * _NC + cid
    base = wid * _RPW

    pltpu.make_async_copy(val_hbm, val_v, sem_b).start()
    pltpu.make_async_copy(bias_hbm, bias_v, sem_b).start()
    for i in range(_NBUF):
        sem = sem_b if i == 0 else osems[i]
        for r in range(_G):
            pltpu.make_async_copy(
                bias_hbm, bufs[i].at[pl.ds(r * _M, _M)], sem).start()
    pltpu.sync_copy(ind_hbm, ind_v)

    def mk_idx(r, _):
        rb = (base + r) * _N
        for kk in range(_ECH):
            src_c = ind_v[0, pl.ds(kk * 16, 16)]
            idx_v[pl.ds(r * _E + kk * 16, 16)] = src_c + jnp.full(
                (16,), rb, jnp.int32)
        return 0

    lax.fori_loop(0, _RPW, mk_idx, 0)

    def g_copy(off, sem):
        return pltpu.make_async_copy(
            x_hbm.at[idx_v.at[pl.ds(off, _GW)]],
            xg_v.at[pl.ds(off, _GW)], sem)

    def g_fire(g, sem):
        for c in range(_GPW // _GW):
            g_copy(g * _GPW + c * _GW, sem).start()

    def g_drain(sem):
        for c in range(_GPW // _GW):
            g_copy(0, sem).wait()

    for i in range(_NBUF):
        g_fire(i, gsems[i])

    pltpu.make_async_copy(val_hbm, val_v, sem_b).wait()
    pltpu.make_async_copy(bias_hbm, bias_v, sem_b).wait()
    for r in range(_G):
        pltpu.make_async_copy(
            bias_hbm, bufs[0].at[pl.ds(0, _M)], sem_b).wait()

    for kk in range(_ECH):
        dst_c = ind_v[1, pl.ds(kk * 16, 16)]
        biasd_v[pl.ds(kk * 16, 16)] = plsc.load_gather(bias_v, [dst_c])

    def scatter_group(buf, g):
        for k in range(_GCH):
            rl = k // _ECH
            kk = k % _ECH
            fc = ind_v[1, pl.ds(kk * 16, 16)] + jnp.full(
                (16,), rl * _M, jnp.int32)
            bd = biasd_v[pl.ds(kk * 16, 16)]
            cc = (xg_v[pl.ds(g * _GPW + k * 16, 16)]
                  * val_v[pl.ds(kk * 16, 16)])
            plsc.store_scatter(buf, [fc], bd)
            plsc.addupdate_scatter(buf, [fc], cc)

    def ring_step(it, _):
        for i in range(_NBUF):
            g = _NBUF * it + i

            @pl.when(it > 0)
            def _wait_out():
                pltpu.make_async_copy(
                    bufs[i], out_hbm.at[pl.ds(base * _M, _G * _M)],
                    osems[i]).wait()

            if i > 0:
                @pl.when(it == 0)
                def _wait_fills():
                    for r in range(_G):
                        pltpu.make_async_copy(
                            bias_hbm, bufs[i].at[pl.ds(0, _M)],
                            osems[i]).wait()

            g_drain(gsems[i])
            scatter_group(bufs[i], g)
            pltpu.make_async_copy(
                bufs[i], out_hbm.at[pl.ds((base + g * _G) * _M, _G * _M)],
                osems[i]).start()

            @pl.when(it < _NIT - 1)
            def _prefetch():
                g_fire(g + _NBUF, gsems[i])

        return 0

    lax.fori_loop(0, _NIT, ring_step, 0)

    for i in range(_NBUF):
        pltpu.make_async_copy(
            bufs[i], out_hbm.at[pl.ds(base * _M, _G * _M)], osems[i]).wait()


@functools.partial(jax.jit, static_argnames=())
def _sc_sparse_linear(x_flat, indices, values, bias_flat):
    mesh = plsc.VectorSubcoreMesh(core_axis_name="c", subcore_axis_name="s")
    kfn = functools.partial(
        pl.kernel, mesh=mesh,
        compiler_params=pltpu.CompilerParams(needs_layout_passes=False),
        out_type=jax.ShapeDtypeStruct((_B * _M,), jnp.float32),
        scratch_types=[
            pltpu.VMEM((2, _E), jnp.int32),
            pltpu.VMEM((_E,), jnp.float32),
            pltpu.VMEM((_M,), jnp.float32),
            pltpu.VMEM((_E,), jnp.float32),
            pltpu.VMEM((_RPW * _E,), jnp.int32),
            pltpu.VMEM((_RPW * _E,), jnp.float32),
            pltpu.VMEM((_G * _M,), jnp.float32),
            pltpu.VMEM((_G * _M,), jnp.float32),
            pltpu.VMEM((_G * _M,), jnp.float32),
            pltpu.VMEM((_G * _M,), jnp.float32),
            pltpu.SemaphoreType.DMA,
            pltpu.SemaphoreType.DMA,
            pltpu.SemaphoreType.DMA,
            pltpu.SemaphoreType.DMA,
            pltpu.SemaphoreType.DMA,
            pltpu.SemaphoreType.DMA,
            pltpu.SemaphoreType.DMA,
            pltpu.SemaphoreType.DMA,
            pltpu.SemaphoreType.DMA,
        ],
    )(_body)
    return kfn(x_flat, indices, values, bias_flat)


def kernel(x, indices, values, bias):
    Bn, Nn, _ = x.shape
    Mn = bias.shape[0]
    x_flat = x.reshape(Bn * Nn)
    out = _sc_sparse_linear(
        x_flat, indices.astype(jnp.int32), values, bias.reshape(Mn))
    return out.reshape(Bn, Mn, 1)


# speedup vs baseline: 1.1394x; 1.1110x over previous
import functools

import jax
import jax.numpy as jnp
from jax import lax
from jax.experimental import pallas as pl
from jax.experimental.pallas import tpu as pltpu
from jax.experimental.pallas import tpu_sc as plsc

_N = 4096
_M = 4096
_E = 64
_B = 8192

_NC = 2
_NS = 16
_NW = _NC * _NS
_RPW = _B // _NW
_G = 4
_NBUF = 2
_NGP = _RPW // _G
_NIT = _NGP // _NBUF
_ECH = _E // 16
_GCH = _G * _ECH
_GW = 128
_GPW = _G * _E


def _body(x_hbm, ind_hbm, val_hbm, bias_hbm, out_hbm,
          ind_v, val_v, bias_v, biasd_v, idx_v, xg_v,
          b0, b1, os0, os1, gs0, gs1, sem_b):
    bufs = [b0, b1]
    osems = [os0, os1]
    gsems = [gs0, gs1]
    cid = lax.axis_index("c")
    sid = lax.axis_index("s")
    wid = sid * _NC + cid
    base = wid * _RPW

    pltpu.make_async_copy(val_hbm, val_v, sem_b).start()
    pltpu.make_async_copy(bias_hbm, bias_v, sem_b).start()
    for i in range(_NBUF):
        sem = sem_b if i == 0 else osems[i]
        for r in range(_G):
            pltpu.make_async_copy(
                bias_hbm, bufs[i].at[pl.ds(r * _M, _M)], sem).start()
    pltpu.sync_copy(ind_hbm, ind_v)

    def mk_idx(r, _):
        rb = (base + r) * _N
        for kk in range(_ECH):
            src_c = ind_v[0, pl.ds(kk * 16, 16)]
            idx_v[pl.ds(r * _E + kk * 16, 16)] = src_c + jnp.full(
                (16,), rb, jnp.int32)
        return 0

    lax.fori_loop(0, _RPW, mk_idx, 0)

    def g_copy(off, sem):
        return pltpu.make_async_copy(
            x_hbm.at[idx_v.at[pl.ds(off, _GW)]],
            xg_v.at[pl.ds(off, _GW)], sem)

    def g_fire(g, sem):
        for c in range(_GPW // _GW):
            g_copy(g * _GPW + c * _GW, sem).start()

    def g_drain(sem):
        for c in range(_GPW // _GW):
            g_copy(0, sem).wait()

    for i in range(_NBUF):
        g_fire(i, gsems[i])

    pltpu.make_async_copy(val_hbm, val_v, sem_b).wait()
    pltpu.make_async_copy(bias_hbm, bias_v, sem_b).wait()
    for r in range(_G):
        pltpu.make_async_copy(
            bias_hbm, bufs[0].at[pl.ds(0, _M)], sem_b).wait()

    for kk in range(_ECH):
        dst_c = ind_v[1, pl.ds(kk * 16, 16)]
        biasd_v[pl.ds(kk * 16, 16)] = plsc.load_gather(bias_v, [dst_c])

    def scatter_group(buf, g):
        for k in range(_GCH):
            rl = k // _ECH
            kk = k % _ECH
            fc = ind_v[1, pl.ds(kk * 16, 16)] + jnp.full(
                (16,), rl * _M, jnp.int32)
            bd = biasd_v[pl.ds(kk * 16, 16)]
            cc = (xg_v[pl.ds(g * _GPW + k * 16, 16)]
                  * val_v[pl.ds(kk * 16, 16)])
            plsc.store_scatter(buf, [fc], bd)
            plsc.addupdate_scatter(buf, [fc], cc)

    def ring_step(it, _):
        for i in range(_NBUF):
            g = _NBUF * it + i

            @pl.when(it > 0)
            def _wait_out():
                pltpu.make_async_copy(
                    bufs[i], out_hbm.at[pl.ds(base * _M, _G * _M)],
                    osems[i]).wait()

            if i > 0:
                @pl.when(it == 0)
                def _wait_fills():
                    for r in range(_G):
                        pltpu.make_async_copy(
                            bias_hbm, bufs[i].at[pl.ds(0, _M)],
                            osems[i]).wait()

            g_drain(gsems[i])
            scatter_group(bufs[i], g)
            pltpu.make_async_copy(
                bufs[i], out_hbm.at[pl.ds((base + g * _G) * _M, _G * _M)],
                osems[i]).start()

            @pl.when(it < _NIT - 1)
            def _prefetch():
                g_fire(g + _NBUF, gsems[i])

        return 0

    lax.fori_loop(0, _NIT, ring_step, 0)

    for i in range(_NBUF):
        pltpu.make_async_copy(
            bufs[i], out_hbm.at[pl.ds(base * _M, _G * _M)], osems[i]).wait()


@functools.partial(jax.jit, static_argnames=())
def _sc_sparse_linear(x_flat, indices, values, bias_flat):
    mesh = plsc.VectorSubcoreMesh(core_axis_name="c", subcore_axis_name="s")
    kfn = functools.partial(
        pl.kernel, mesh=mesh,
        compiler_params=pltpu.CompilerParams(needs_layout_passes=False),
        out_type=jax.ShapeDtypeStruct((_B * _M,), jnp.float32),
        scratch_types=[
            pltpu.VMEM((2, _E), jnp.int32),
            pltpu.VMEM((_E,), jnp.float32),
            pltpu.VMEM((_M,), jnp.float32),
            pltpu.VMEM((_E,), jnp.float32),
            pltpu.VMEM((_RPW * _E,), jnp.int32),
            pltpu.VMEM((_RPW * _E,), jnp.float32),
            pltpu.VMEM((_G * _M,), jnp.float32),
            pltpu.VMEM((_G * _M,), jnp.float32),
            pltpu.SemaphoreType.DMA,
            pltpu.SemaphoreType.DMA,
            pltpu.SemaphoreType.DMA,
            pltpu.SemaphoreType.DMA,
            pltpu.SemaphoreType.DMA,
        ],
    )(_body)
    return kfn(x_flat, indices, values, bias_flat)


def kernel(x, indices, values, bias):
    Bn, Nn, _ = x.shape
    Mn = bias.shape[0]
    x_flat = x.reshape(Bn * Nn)
    out = _sc_sparse_linear(
        x_flat, indices.astype(jnp.int32), values, bias.reshape(Mn))
    return out.reshape(Bn, Mn, 1)
